# baseline (speedup 1.0000x reference)
import functools

import jax
import jax.numpy as jnp
from jax import lax
from jax.experimental import pallas as pl
from jax.experimental.pallas import tpu as pltpu
from jax.experimental.pallas import tpu_sc as plsc

N = 10000
E = 320000
D = 128
NG = 64

NC = 2
NS = 16
NW = NC * NS

EB = 128
NBLK = 80
E_PAD = NW * NBLK * EB
N_PAD = 10240
ROWS = N_PAD // NS

_mesh = plsc.VectorSubcoreMesh(core_axis_name="c", subcore_axis_name="s")


NBUF = 4
QTR = 16
SEGS = NBLK // QTR
NGRP = 2 * QTR // NBUF


@functools.partial(
    pl.kernel,
    out_type=jax.ShapeDtypeStruct((NC * N_PAD, D), jnp.float32),
    mesh=_mesh,
    scratch_types=[
        pltpu.VMEM((QTR, EB), jnp.int32),
        pltpu.VMEM((QTR, EB), jnp.int32),
        pltpu.VMEM((EB // 2, D), jnp.float32),
        pltpu.VMEM((EB // 2, D), jnp.float32),
        pltpu.VMEM((EB // 2, D), jnp.float32),
        pltpu.VMEM((EB // 2, D), jnp.float32),
        pltpu.VMEM_SHARED((N_PAD, D), jnp.float32),
    ] + [pltpu.SemaphoreType.DMA] * (2 * NBUF),
)
def _sc_propagate(h_hbm, src_hbm, dst_hbm, zeros_hbm, out, src_v, dst_v,
                  rows0, rows1, rows2, rows3, acc, g0, g1, g2, g3,
                  s0, s1, s2, s3):
    gsem = (g0, g1, g2, g3)
    ssem = (s0, s1, s2, s3)
    rowsb = (rows0, rows1, rows2, rows3)
    HB = EB // 2
    c = lax.axis_index("c")
    s = lax.axis_index("s")
    wid = c * NS + s
    r0 = s * ROWS
    pltpu.sync_copy(zeros_hbm, acc.at[pl.ds(r0, ROWS)])
    plsc.subcore_barrier()

    def idx_view(ref, j2):
        return ref.at[j2 // 2, pl.ds((j2 % 2) * HB, HB)]

    def fire_gather(j2, b):
        pltpu.async_copy(h_hbm.at[idx_view(src_v, j2)], rowsb[b], gsem[b])

    def wait_gather(j2, b):
        pltpu.make_async_copy(h_hbm.at[idx_view(src_v, j2)], rowsb[b],
                              gsem[b]).wait()

    def wait_scatter(j2, b):
        pltpu.make_async_copy(rowsb[b], acc.at[idx_view(dst_v, j2)],
                              ssem[b]).wait()

    def group(g, fire_next):
        base = g * NBUF
        for b in range(NBUF):
            wait_gather(base + b, b)
            pltpu.async_copy(rowsb[b], acc.at[idx_view(dst_v, base + b)],
                             ssem[b], add=True)
        for b in range(NBUF):
            wait_scatter(base + b, b)
            if fire_next:
                fire_gather(base + NBUF + b, b)

    def body(g, carry):
        group(g, True)
        return carry

    for q in range(SEGS):
        blk0 = wid * NBLK + q * QTR
        pltpu.sync_copy(src_hbm.at[pl.ds(blk0, QTR)], src_v)
        pltpu.sync_copy(dst_hbm.at[pl.ds(blk0, QTR)], dst_v)
        for b in range(NBUF):
            fire_gather(b, b)
        lax.fori_loop(0, NGRP - 1, body, 0)
        group(NGRP - 1, False)

    plsc.subcore_barrier()
    pltpu.sync_copy(acc.at[pl.ds(r0, ROWS)], out.at[pl.ds(c * N_PAD + r0, ROWS)])


DBLK = E_PAD // (NS * EB)
DSEG = DBLK // QTR


@functools.partial(
    pl.kernel,
    out_type=jax.ShapeDtypeStruct((NC * N_PAD, D), jnp.float32),
    mesh=_mesh,
    scratch_types=[
        pltpu.VMEM((QTR, EB), jnp.int32),
        pltpu.VMEM((EB, D), jnp.float32),
        pltpu.VMEM_SHARED((N_PAD, D), jnp.float32),
    ],
)
def _sc_degree(dst2_hbm, ones_hbm, zeros_hbm, out, dst_v, ones_v, acc):
    c = lax.axis_index("c")
    s = lax.axis_index("s")
    r0 = s * ROWS
    pltpu.sync_copy(zeros_hbm, acc.at[pl.ds(r0, ROWS)])
    pltpu.sync_copy(ones_hbm, ones_v)
    plsc.subcore_barrier()

    def body(j, carry):
        pltpu.sync_copy(ones_v, acc.at[dst_v.at[j]], add=True)
        return carry

    for q in range(DSEG):
        blk0 = (c * NS + s) * DBLK + q * QTR
        pltpu.sync_copy(dst2_hbm.at[pl.ds(blk0, QTR)], dst_v)
        lax.fori_loop(0, QTR, body, 0)

    plsc.subcore_barrier()
    pltpu.sync_copy(acc.at[pl.ds(r0, ROWS)], out.at[pl.ds(c * N_PAD + r0, ROWS)])


def _tc(body, out_shape, *args):
    return pl.pallas_call(body, out_shape=out_shape)(*args)


def _dinv_body(hist_ref, o_ref):
    rid = lax.broadcasted_iota(jnp.int32, (N_PAD, 1), 0)
    for l in range(2):
        deg = hist_ref[l, :, 0:1] + 1.0
        o_ref[l] = jnp.where(rid < N, lax.rsqrt(deg), 0.0)


def _first_body(x_ref, w_ref, dinv_ref, o_ref):
    o_ref[...] = jnp.dot(x_ref[...], w_ref[...],
                         preferred_element_type=jnp.float32) * dinv_ref[...]


def _mid_body(part_ref, hp_ref, dinv_ref, b_ref, w_ref, o_ref):
    dinv = dinv_ref[...]
    acc = part_ref[0:N_PAD, :] + part_ref[N_PAD:2 * N_PAD, :] + hp_ref[...]
    feat = jnp.maximum(dinv * acc + b_ref[...], 0.0)
    o_ref[...] = jnp.dot(feat, w_ref[...],
                         preferred_element_type=jnp.float32) * dinv


def _selout_body(part_ref, hp_ref, dinv_ref, b_ref, fcw_ref, fcb_ref,
                 x_ref, w1_ref, p_ref, h_ref):
    dinv = dinv_ref[...]
    acc = part_ref[0:N_PAD, :] + part_ref[N_PAD:2 * N_PAD, :] + hp_ref[...]
    feat = jnp.maximum(dinv * acc + b_ref[...], 0.0)
    logit = jnp.dot(feat, fcw_ref[...],
                    preferred_element_type=jnp.float32) + fcb_ref[...]
    p = 1.0 / (1.0 + jnp.exp(-logit))
    p_ref[...] = p
    h_ref[...] = jnp.dot(x_ref[...] * p, w1_ref[...],
                         preferred_element_type=jnp.float32) * dinv


def _pool_body(part_ref, hp_ref, dinv_ref, b_ref, batch_ref, o_ref):
    dinv = dinv_ref[...]
    acc = part_ref[0:N_PAD, :] + part_ref[N_PAD:2 * N_PAD, :] + hp_ref[...]
    feat = jnp.maximum(dinv * acc + b_ref[...], 0.0)
    gid = lax.broadcasted_iota(jnp.int32, (NG, 1), 0)
    mask = (batch_ref[...] == gid).astype(jnp.float32)
    ssum = jnp.dot(mask, feat, preferred_element_type=jnp.float32)
    cnt = jnp.sum(mask, axis=1, keepdims=True)
    o_ref[...] = ssum / jnp.maximum(cnt, 1.0)


def _pad_rows(x):
    return jnp.pad(x, ((0, N_PAD - N), (0, 0)))


def _edges(edge_index):
    src = edge_index[0].astype(jnp.int32)
    dst = edge_index[1].astype(jnp.int32)
    pad = jnp.full((E_PAD - E,), N, jnp.int32)
    src = jnp.concatenate([src, pad]).reshape(NW * NBLK, EB)
    dst = jnp.concatenate([dst, pad]).reshape(NW * NBLK, EB)
    return src, dst


def kernel(x_H, x_G, edge_index_H, edge_index_G, batch_H, batch_G,
           sel_W1, sel_b1, sel_W2, sel_b2, sel_fcW, sel_fcb,
           emb_W1, emb_b1, emb_W2, emb_b2):
    f32 = jnp.float32
    x_Hp = _pad_rows(x_H)
    x_Gp = _pad_rows(x_G)
    srcH, dstH = _edges(edge_index_H)
    srcG, dstG = _edges(edge_index_G)
    batch_Hp = jnp.concatenate(
        [batch_H.astype(jnp.int32), jnp.full((N_PAD - N,), -1, jnp.int32)]).reshape(1, N_PAD)
    batch_Gp = jnp.concatenate(
        [batch_G.astype(jnp.int32), jnp.full((N_PAD - N,), -1, jnp.int32)]).reshape(1, N_PAD)
    onesEB = jnp.ones((EB, D), f32)
    zerosD = jnp.zeros((ROWS, D), f32)
    b1s = sel_b1.reshape(1, D)
    b2s = sel_b2.reshape(1, D)
    fcb = sel_fcb.reshape(1, 1)
    b1e = emb_b1.reshape(1, D)
    b2e = emb_b2.reshape(1, D)

    def prop(h, src, dst):
        return _sc_propagate(h, src, dst, zerosD)

    hist2 = _sc_degree(jnp.concatenate([dstH, dstG], axis=0), onesEB, zerosD)
    dinv2 = _tc(_dinv_body, jax.ShapeDtypeStruct((2, N_PAD, 1), f32),
                hist2.reshape(NC, N_PAD, D))
    dinv_H = dinv2[0]
    dinv_G = dinv2[1]

    sd = jax.ShapeDtypeStruct((N_PAD, D), f32)

    h1 = _tc(_first_body, sd, x_Hp, sel_W1, dinv_H)
    a1 = prop(h1, srcH, dstH)
    h2 = _tc(_mid_body, sd, a1, h1, dinv_H, b1s, sel_W2)
    a2 = prop(h2, srcH, dstH)
    p_pad, h3 = _tc(
        _selout_body,
        (jax.ShapeDtypeStruct((N_PAD, 1), f32), sd),
        a2, h2, dinv_H, b2s, sel_fcW, fcb, x_Hp, emb_W1)

    a3 = prop(h3, srcH, dstH)
    h4 = _tc(_mid_body, sd, a3, h3, dinv_H, b1e, emb_W2)
    a4 = prop(h4, srcH, dstH)
    h_F = _tc(_pool_body, jax.ShapeDtypeStruct((NG, D), f32),
              a4, h4, dinv_H, b2e, batch_Hp)

    g1 = _tc(_first_body, sd, x_Gp, emb_W1, dinv_G)
    c1 = prop(g1, srcG, dstG)
    g2 = _tc(_mid_body, sd, c1, g1, dinv_G, b1e, emb_W2)
    c2 = prop(g2, srcG, dstG)
    h_G = _tc(_pool_body, jax.ShapeDtypeStruct((NG, D), f32),
              c2, g2, dinv_G, b2e, batch_Gp)

    return (h_F, h_G, p_pad[:N])


# speedup vs baseline: 1.0238x; 1.0238x over previous
import functools

import jax
import jax.numpy as jnp
from jax import lax
from jax.experimental import pallas as pl
from jax.experimental.pallas import tpu as pltpu
from jax.experimental.pallas import tpu_sc as plsc

N = 10000
E = 320000
D = 128
NG = 64

NC = 2
NS = 16
NW = NC * NS

EB = 128
NBLK = 80
E_PAD = NW * NBLK * EB
N_PAD = 10240
ROWS = N_PAD // NS

_mesh = plsc.VectorSubcoreMesh(core_axis_name="c", subcore_axis_name="s")


NBUF = 4
QTR = 16
SEGS = NBLK // QTR
NGRP = 2 * QTR // NBUF


@functools.partial(
    pl.kernel,
    out_type=jax.ShapeDtypeStruct((NC * N_PAD, D), jnp.float32),
    mesh=_mesh,
    scratch_types=[
        pltpu.VMEM((QTR, EB), jnp.int32),
        pltpu.VMEM((QTR, EB), jnp.int32),
        pltpu.VMEM((EB // 2, D), jnp.float32),
        pltpu.VMEM((EB // 2, D), jnp.float32),
        pltpu.VMEM((EB // 2, D), jnp.float32),
        pltpu.VMEM((EB // 2, D), jnp.float32),
        pltpu.VMEM_SHARED((N_PAD, D), jnp.float32),
    ] + [pltpu.SemaphoreType.DMA] * NBUF,
)
def _sc_propagate(h_hbm, src_hbm, dst_hbm, zeros_hbm, out, src_v, dst_v,
                  rows0, rows1, rows2, rows3, acc, g0, g1, g2, g3):
    gsem = (g0, g1, g2, g3)
    rowsb = (rows0, rows1, rows2, rows3)
    HB = EB // 2
    c = lax.axis_index("c")
    s = lax.axis_index("s")
    wid = c * NS + s
    r0 = s * ROWS
    pltpu.sync_copy(zeros_hbm, acc.at[pl.ds(r0, ROWS)])
    plsc.subcore_barrier()

    def idx_view(ref, j2):
        return ref.at[j2 // 2, pl.ds((j2 % 2) * HB, HB)]

    def fire_gather(j2, b):
        pltpu.async_copy(h_hbm.at[idx_view(src_v, j2)], rowsb[b], gsem[b])

    def wait_gather(j2, b):
        pltpu.make_async_copy(h_hbm.at[idx_view(src_v, j2)], rowsb[b],
                              gsem[b]).wait()

    def group(g, fire_next):
        base = g * NBUF
        for b in range(NBUF):
            wait_gather(base + b, b)
            pltpu.sync_copy(rowsb[b], acc.at[idx_view(dst_v, base + b)],
                            add=True)
            if fire_next:
                fire_gather(base + NBUF + b, b)

    def body(g, carry):
        group(g, True)
        return carry

    for q in range(SEGS):
        blk0 = wid * NBLK + q * QTR
        pltpu.sync_copy(src_hbm.at[pl.ds(blk0, QTR)], src_v)
        pltpu.sync_copy(dst_hbm.at[pl.ds(blk0, QTR)], dst_v)
        for b in range(NBUF):
            fire_gather(b, b)
        lax.fori_loop(0, NGRP - 1, body, 0)
        group(NGRP - 1, False)

    plsc.subcore_barrier()
    pltpu.sync_copy(acc.at[pl.ds(r0, ROWS)], out.at[pl.ds(c * N_PAD + r0, ROWS)])


DBLK = E_PAD // (NS * EB)
DSEG = DBLK // QTR


@functools.partial(
    pl.kernel,
    out_type=jax.ShapeDtypeStruct((NC * N_PAD, D), jnp.float32),
    mesh=_mesh,
    scratch_types=[
        pltpu.VMEM((QTR, EB), jnp.int32),
        pltpu.VMEM((EB, D), jnp.float32),
        pltpu.VMEM_SHARED((N_PAD, D), jnp.float32),
    ],
)
def _sc_degree(dst2_hbm, ones_hbm, zeros_hbm, out, dst_v, ones_v, acc):
    c = lax.axis_index("c")
    s = lax.axis_index("s")
    r0 = s * ROWS
    pltpu.sync_copy(zeros_hbm, acc.at[pl.ds(r0, ROWS)])
    pltpu.sync_copy(ones_hbm, ones_v)
    plsc.subcore_barrier()

    def body(j, carry):
        pltpu.sync_copy(ones_v, acc.at[dst_v.at[j]], add=True)
        return carry

    for q in range(DSEG):
        blk0 = (c * NS + s) * DBLK + q * QTR
        pltpu.sync_copy(dst2_hbm.at[pl.ds(blk0, QTR)], dst_v)
        lax.fori_loop(0, QTR, body, 0)

    plsc.subcore_barrier()
    pltpu.sync_copy(acc.at[pl.ds(r0, ROWS)], out.at[pl.ds(c * N_PAD + r0, ROWS)])


def _tc(body, out_shape, *args):
    return pl.pallas_call(body, out_shape=out_shape)(*args)


def _dinv_body(hist_ref, o_ref):
    rid = lax.broadcasted_iota(jnp.int32, (N_PAD, 1), 0)
    for l in range(2):
        deg = hist_ref[l, :, 0:1] + 1.0
        o_ref[l] = jnp.where(rid < N, lax.rsqrt(deg), 0.0)


def _first_body(x_ref, w_ref, dinv_ref, o_ref):
    o_ref[...] = jnp.dot(x_ref[...], w_ref[...],
                         preferred_element_type=jnp.float32) * dinv_ref[...]


def _mid_body(part_ref, hp_ref, dinv_ref, b_ref, w_ref, o_ref):
    dinv = dinv_ref[...]
    acc = part_ref[0:N_PAD, :] + part_ref[N_PAD:2 * N_PAD, :] + hp_ref[...]
    feat = jnp.maximum(dinv * acc + b_ref[...], 0.0)
    o_ref[...] = jnp.dot(feat, w_ref[...],
                         preferred_element_type=jnp.float32) * dinv


def _selout_body(part_ref, hp_ref, dinv_ref, b_ref, fcw_ref, fcb_ref,
                 x_ref, w1_ref, p_ref, h_ref):
    dinv = dinv_ref[...]
    acc = part_ref[0:N_PAD, :] + part_ref[N_PAD:2 * N_PAD, :] + hp_ref[...]
    feat = jnp.maximum(dinv * acc + b_ref[...], 0.0)
    logit = jnp.dot(feat, fcw_ref[...],
                    preferred_element_type=jnp.float32) + fcb_ref[...]
    p = 1.0 / (1.0 + jnp.exp(-logit))
    p_ref[...] = p
    h_ref[...] = jnp.dot(x_ref[...] * p, w1_ref[...],
                         preferred_element_type=jnp.float32) * dinv


def _pool_body(part_ref, hp_ref, dinv_ref, b_ref, batch_ref, o_ref):
    dinv = dinv_ref[...]
    acc = part_ref[0:N_PAD, :] + part_ref[N_PAD:2 * N_PAD, :] + hp_ref[...]
    feat = jnp.maximum(dinv * acc + b_ref[...], 0.0)
    gid = lax.broadcasted_iota(jnp.int32, (NG, 1), 0)
    mask = (batch_ref[...] == gid).astype(jnp.float32)
    ssum = jnp.dot(mask, feat, preferred_element_type=jnp.float32)
    cnt = jnp.sum(mask, axis=1, keepdims=True)
    o_ref[...] = ssum / jnp.maximum(cnt, 1.0)


def _pad_rows(x):
    return jnp.pad(x, ((0, N_PAD - N), (0, 0)))


def _edges(edge_index):
    src = edge_index[0].astype(jnp.int32)
    dst = edge_index[1].astype(jnp.int32)
    pad = jnp.full((E_PAD - E,), N, jnp.int32)
    src = jnp.concatenate([src, pad]).reshape(NW * NBLK, EB)
    dst = jnp.concatenate([dst, pad]).reshape(NW * NBLK, EB)
    return src, dst


def kernel(x_H, x_G, edge_index_H, edge_index_G, batch_H, batch_G,
           sel_W1, sel_b1, sel_W2, sel_b2, sel_fcW, sel_fcb,
           emb_W1, emb_b1, emb_W2, emb_b2):
    f32 = jnp.float32
    x_Hp = _pad_rows(x_H)
    x_Gp = _pad_rows(x_G)
    srcH, dstH = _edges(edge_index_H)
    srcG, dstG = _edges(edge_index_G)
    batch_Hp = jnp.concatenate(
        [batch_H.astype(jnp.int32), jnp.full((N_PAD - N,), -1, jnp.int32)]).reshape(1, N_PAD)
    batch_Gp = jnp.concatenate(
        [batch_G.astype(jnp.int32), jnp.full((N_PAD - N,), -1, jnp.int32)]).reshape(1, N_PAD)
    onesEB = jnp.ones((EB, D), f32)
    zerosD = jnp.zeros((ROWS, D), f32)
    b1s = sel_b1.reshape(1, D)
    b2s = sel_b2.reshape(1, D)
    fcb = sel_fcb.reshape(1, 1)
    b1e = emb_b1.reshape(1, D)
    b2e = emb_b2.reshape(1, D)

    def prop(h, src, dst):
        return _sc_propagate(h, src, dst, zerosD)

    hist2 = _sc_degree(jnp.concatenate([dstH, dstG], axis=0), onesEB, zerosD)
    dinv2 = _tc(_dinv_body, jax.ShapeDtypeStruct((2, N_PAD, 1), f32),
                hist2.reshape(NC, N_PAD, D))
    dinv_H = dinv2[0]
    dinv_G = dinv2[1]

    sd = jax.ShapeDtypeStruct((N_PAD, D), f32)

    h1 = _tc(_first_body, sd, x_Hp, sel_W1, dinv_H)
    a1 = prop(h1, srcH, dstH)
    h2 = _tc(_mid_body, sd, a1, h1, dinv_H, b1s, sel_W2)
    a2 = prop(h2, srcH, dstH)
    p_pad, h3 = _tc(
        _selout_body,
        (jax.ShapeDtypeStruct((N_PAD, 1), f32), sd),
        a2, h2, dinv_H, b2s, sel_fcW, fcb, x_Hp, emb_W1)

    a3 = prop(h3, srcH, dstH)
    h4 = _tc(_mid_body, sd, a3, h3, dinv_H, b1e, emb_W2)
    a4 = prop(h4, srcH, dstH)
    h_F = _tc(_pool_body, jax.ShapeDtypeStruct((NG, D), f32),
              a4, h4, dinv_H, b2e, batch_Hp)

    g1 = _tc(_first_body, sd, x_Gp, emb_W1, dinv_G)
    c1 = prop(g1, srcG, dstG)
    g2 = _tc(_mid_body, sd, c1, g1, dinv_G, b1e, emb_W2)
    c2 = prop(g2, srcG, dstG)
    h_G = _tc(_pool_body, jax.ShapeDtypeStruct((NG, D), f32),
              c2, g2, dinv_G, b2e, batch_Gp)

    return (h_F, h_G, p_pad[:N])
